# baseline (speedup 1.0000x reference)
import functools

import jax
import jax.numpy as jnp
from jax import lax
from jax.experimental import pallas as pl
from jax.experimental.pallas import tpu as pltpu
from jax.experimental.pallas import tpu_sc as plsc

MARGIN1 = 0.4
MARGIN2 = 0.4

NC = 2
NS = 16
LANES = 16
NW = NC * NS


def _make_lse(B, C, RB):
    def body(ct_ref, o_ref, flat_ref):
        x = ct_ref[...]
        m = jnp.max(x, axis=0)
        s = jnp.sum(jnp.exp(x - m[None, :]), axis=0)
        o_ref[...] = m + jnp.log(s)
        for b in range(RB // 128):
            flat_ref[:, b, :] = x[:, b * 128:(b + 1) * 128]

    return pl.pallas_call(
        body,
        grid=(B // RB,),
        in_specs=[pl.BlockSpec((C, RB), lambda i: (0, i))],
        out_specs=[pl.BlockSpec((RB,), lambda i: (i,)),
                   pl.BlockSpec((C, RB // 128, 128), lambda i: (0, i, 0))],
        out_shape=[jax.ShapeDtypeStruct((B,), jnp.float32),
                   jax.ShapeDtypeStruct((C, B // 128, 128), jnp.float32)],
    )


def _make_sc_dist(B, D, T, K):
    TW = T // NW
    NCHUNK = TW // K
    GROUPS = K // LANES

    mesh = plsc.VectorSubcoreMesh(core_axis_name="c", subcore_axis_name="s")

    DW = D // 2
    buf_t = [
        pltpu.VMEM((K,), jnp.int32),
        pltpu.VMEM((K,), jnp.int32),
        pltpu.VMEM((K,), jnp.int32),
        pltpu.VMEM((K,), jnp.int32),
        pltpu.VMEM((K, DW), jnp.int32),
        pltpu.VMEM((K, DW), jnp.int32),
        pltpu.VMEM((K, DW), jnp.int32),
        pltpu.VMEM((K, DW), jnp.int32),
        pltpu.VMEM((4 * K,), jnp.int32),
        pltpu.VMEM((K,), jnp.float32),
        pltpu.VMEM((K,), jnp.float32),
    ]

    @functools.partial(
        pl.kernel,
        mesh=mesh,
        compiler_params=pltpu.CompilerParams(
            needs_layout_passes=False, use_tc_tiling_on_sc=False),
        out_type=[jax.ShapeDtypeStruct((T,), jnp.float32),
                  jax.ShapeDtypeStruct((T,), jnp.float32),
                  jax.ShapeDtypeStruct((4 * T,), jnp.int32)],
        scratch_types=[
            pltpu.VMEM((B,), jnp.int32),
            *buf_t,
            *buf_t,
            pltpu.SemaphoreType.DMA,
            pltpu.SemaphoreType.DMA,
            pltpu.SemaphoreType.DMA,
            pltpu.SemaphoreType.DMA,
            pltpu.SemaphoreType.DMA,
        ],
    )
    def sc_dist(emb, tidx, tgt, d1_out, d2_out, fidx_out,
                target_v, *rest):
        bufs = (rest[0:11], rest[11:22])
        sem_i = rest[22]
        sem_d = (rest[23], rest[24])
        sem_o = (rest[25], rest[26])

        wid = lax.axis_index("s") * NC + lax.axis_index("c")
        base_t = wid * TW

        pltpu.sync_copy(tgt, target_v)

        zf = jnp.zeros((LANES,), jnp.float32)
        iota = lax.iota(jnp.int32, LANES)
        def fire_idx(ch, p):
            tb = base_t + ch * K
            for q in range(4):
                pltpu.async_copy(
                    tidx.at[q, pl.ds(tb, K)], bufs[p][q], sem_i)

        def drain_idx(p):
            for q in range(4):
                pltpu.make_async_copy(
                    tidx.at[q, pl.ds(0, K)], bufs[p][q], sem_i).wait()

        def fire_rows(p):
            for q in range(4):
                pltpu.async_copy(emb.at[bufs[p][q]], bufs[p][4 + q], sem_d[p])

        def drain_rows(p):
            for q in range(4):
                pltpu.make_async_copy(
                    emb.at[bufs[p][q]], bufs[p][4 + q], sem_d[p]).wait()

        def phase_a(p, drain_fidx):
            idxa_v, _, idxr_v, idxn_v = bufs[p][0:4]
            fidx_v = bufs[p][8]
            if drain_fidx:
                pltpu.make_async_copy(
                    fidx_v, fidx_out.at[pl.ds(0, 4 * K)], sem_o[p]).wait()

            def ga(g, carry):
                gb = g * LANES
                av = idxa_v[pl.ds(gb, LANES)]
                rv = idxr_v[pl.ds(gb, LANES)]
                nv = idxn_v[pl.ds(gb, LANES)]
                ta = plsc.load_gather(target_v, [av])
                tr = plsc.load_gather(target_v, [rv])
                tn = plsc.load_gather(target_v, [nv])
                fidx_v[pl.ds(gb, LANES)] = tr * B + av
                fidx_v[pl.ds(K + gb, LANES)] = ta * B + rv
                fidx_v[pl.ds(2 * K + gb, LANES)] = tn * B + av
                fidx_v[pl.ds(3 * K + gb, LANES)] = ta * B + nv
                return carry

            lax.fori_loop(0, GROUPS, ga, 0)

        def phase_b(ch, p, drain_d):
            ra_v, rp_v, rr_v, rn_v = bufs[p][4:8]
            d1_v, d2_v = bufs[p][9], bufs[p][10]
            if drain_d:
                pltpu.make_async_copy(
                    d1_v, d1_out.at[pl.ds(0, K)], sem_o[p]).wait()
                pltpu.make_async_copy(
                    d2_v, d2_out.at[pl.ds(0, K)], sem_o[p]).wait()

            def gb_fn(g, carry):
                gb = g * LANES
                jv = gb + iota

                def dbody(dd, c):
                    dap, dar, dan, dv = c
                    for _ in range(4):
                        cv = jnp.bitwise_and(dv + iota, DW - 1)
                        wa = plsc.load_gather(ra_v, [jv, cv])
                        wp = plsc.load_gather(rp_v, [jv, cv])
                        wr = plsc.load_gather(rr_v, [jv, cv])
                        wn = plsc.load_gather(rn_v, [jv, cv])
                        for hi in (False, True):
                            if hi:
                                ea = plsc.bitcast(wa & -65536, jnp.float32)
                                ep = plsc.bitcast(wp & -65536, jnp.float32)
                                er = plsc.bitcast(wr & -65536, jnp.float32)
                                en = plsc.bitcast(wn & -65536, jnp.float32)
                            else:
                                ea = plsc.bitcast(wa << 16, jnp.float32)
                                ep = plsc.bitcast(wp << 16, jnp.float32)
                                er = plsc.bitcast(wr << 16, jnp.float32)
                                en = plsc.bitcast(wn << 16, jnp.float32)
                            s1 = ea - ep
                            s2 = ea - er
                            s3 = ea - en
                            dap = dap + s1 * s1
                            dar = dar + s2 * s2
                            dan = dan + s3 * s3
                        dv = dv + 1
                    return (dap, dar, dan, dv)

                dap, dar, dan, _ = lax.fori_loop(
                    0, DW // 4, dbody,
                    (zf, zf, zf, jnp.zeros((LANES,), jnp.int32)))
                d1_v[pl.ds(gb, LANES)] = dap - dar
                d2_v[pl.ds(gb, LANES)] = dar - dan
                return carry

            lax.fori_loop(0, GROUPS, gb_fn, 0)
            tb = base_t + ch * K
            pltpu.async_copy(d1_v, d1_out.at[pl.ds(tb, K)], sem_o[p])
            pltpu.async_copy(d2_v, d2_out.at[pl.ds(tb, K)], sem_o[p])
            pltpu.async_copy(
                bufs[p][8], fidx_out.at[pl.ds(4 * tb, 4 * K)], sem_o[p])

        fire_idx(0, 0)
        drain_idx(0)
        phase_a(0, False)
        fire_rows(0)

        def pair_impl(e, first):
            o = e + 1
            e2 = jnp.minimum(e + 2, NCHUNK - 2)

            fire_idx(o, 1)
            drain_idx(1)
            phase_a(1, not first)
            fire_rows(1)

            drain_rows(0)
            phase_b(e, 0, not first)

            fire_idx(e2, 0)
            drain_idx(0)
            phase_a(0, True)
            fire_rows(0)

            drain_rows(1)
            phase_b(o, 1, not first)

        pair_impl(0, True)

        def pair_body(gp, carry):
            pair_impl(2 * gp, False)
            return carry

        lax.fori_loop(1, NCHUNK // 2, pair_body, 0)
        drain_rows(0)
        pltpu.make_async_copy(
            bufs[0][9], d1_out.at[pl.ds(0, K)], sem_o[0]).wait()
        pltpu.make_async_copy(
            bufs[0][10], d2_out.at[pl.ds(0, K)], sem_o[0]).wait()
        pltpu.make_async_copy(
            bufs[1][8], fidx_out.at[pl.ds(0, 4 * K)], sem_o[1]).wait()
        pltpu.make_async_copy(
            bufs[1][9], d1_out.at[pl.ds(0, K)], sem_o[1]).wait()
        pltpu.make_async_copy(
            bufs[1][10], d2_out.at[pl.ds(0, K)], sem_o[1]).wait()

    return sc_dist


def _make_sc_weight(B, T, K):
    TW = T // NW
    NCHUNK = TW // K
    GROUPS = K // LANES

    mesh = plsc.VectorSubcoreMesh(core_axis_name="c", subcore_axis_name="s")

    buf_t = [
        pltpu.VMEM((4 * K,), jnp.int32),
        pltpu.VMEM((4 * K,), jnp.float32),
        pltpu.VMEM((K,), jnp.int32),
        pltpu.VMEM((K,), jnp.int32),
        pltpu.VMEM((K,), jnp.int32),
        pltpu.VMEM((K,), jnp.float32),
        pltpu.VMEM((K,), jnp.float32),
    ]

    @functools.partial(
        pl.kernel,
        mesh=mesh,
        compiler_params=pltpu.CompilerParams(
            needs_layout_passes=False, use_tc_tiling_on_sc=False),
        out_type=jax.ShapeDtypeStruct((NW, LANES), jnp.float32),
        scratch_types=[
            pltpu.VMEM((B,), jnp.float32),
            *buf_t,
            *buf_t,
            pltpu.VMEM((LANES,), jnp.float32),
            pltpu.SemaphoreType.DMA,
            pltpu.SemaphoreType.DMA,
        ],
    )
    def sc_weight(tidx, lse, conf_flat, fidx_in, d1_in, d2_in,
                  out, lse_v, *rest):
        bufs = (rest[0:7], rest[7:14])
        accv = rest[14]
        sem = (rest[15], rest[16])

        wid = lax.axis_index("s") * NC + lax.axis_index("c")
        base_t = wid * TW

        pltpu.sync_copy(lse, lse_v)

        zf = jnp.zeros((LANES,), jnp.float32)

        def fire(ch, p):
            tb = base_t + ch * K
            b = bufs[p]
            pltpu.async_copy(fidx_in.at[pl.ds(4 * tb, 4 * K)], b[0], sem[p])
            pltpu.async_copy(tidx.at[0, pl.ds(tb, K)], b[2], sem[p])
            pltpu.async_copy(tidx.at[2, pl.ds(tb, K)], b[3], sem[p])
            pltpu.async_copy(tidx.at[3, pl.ds(tb, K)], b[4], sem[p])
            pltpu.async_copy(d1_in.at[pl.ds(tb, K)], b[5], sem[p])
            pltpu.async_copy(d2_in.at[pl.ds(tb, K)], b[6], sem[p])

        def drain_pre(p):
            b = bufs[p]
            pltpu.make_async_copy(
                fidx_in.at[pl.ds(0, 4 * K)], b[0], sem[p]).wait()
            pltpu.make_async_copy(
                tidx.at[0, pl.ds(0, K)], b[2], sem[p]).wait()
            pltpu.make_async_copy(
                tidx.at[2, pl.ds(0, K)], b[3], sem[p]).wait()
            pltpu.make_async_copy(
                tidx.at[3, pl.ds(0, K)], b[4], sem[p]).wait()
            pltpu.make_async_copy(d1_in.at[pl.ds(0, K)], b[5], sem[p]).wait()
            pltpu.make_async_copy(d2_in.at[pl.ds(0, K)], b[6], sem[p]).wait()

        def fire_cval(p):
            b = bufs[p]
            pltpu.async_copy(conf_flat.at[b[0]], b[1], sem[p])

        def drain_cval(p):
            b = bufs[p]
            pltpu.make_async_copy(conf_flat.at[b[0]], b[1], sem[p]).wait()

        def compute(p, acc):
            b = bufs[p]
            cval_v, idxa_v, idxr_v, idxn_v, d1_v, d2_v = b[1:7]

            def gb_fn(g, acc_in):
                gb = g * LANES
                av = idxa_v[pl.ds(gb, LANES)]
                rv = idxr_v[pl.ds(gb, LANES)]
                nv = idxn_v[pl.ds(gb, LANES)]
                la = plsc.load_gather(lse_v, [av])
                lr = plsc.load_gather(lse_v, [rv])
                ln = plsc.load_gather(lse_v, [nv])
                c1 = cval_v[pl.ds(gb, LANES)]
                c2 = cval_v[pl.ds(K + gb, LANES)]
                c3 = cval_v[pl.ds(2 * K + gb, LANES)]
                c4 = cval_v[pl.ds(3 * K + gb, LANES)]
                w_rel = jnp.exp(jnp.exp(c1 - la) + jnp.exp(c2 - lr))
                w_neg = jnp.exp(jnp.exp(c3 - la) + jnp.exp(c4 - ln))
                d1 = d1_v[pl.ds(gb, LANES)]
                d2 = d2_v[pl.ds(gb, LANES)]
                loss = (jnp.maximum(d1 + w_rel * MARGIN1, 0.0)
                        + jnp.maximum(d2 + w_neg * MARGIN2, 0.0))
                return acc_in + loss

            return lax.fori_loop(0, GROUPS, gb_fn, acc)

        fire(0, 0)
        drain_pre(0)
        fire_cval(0)

        def pair_body(gp, acc):
            e = 2 * gp
            o = e + 1
            e2 = jnp.minimum(e + 2, NCHUNK - 2)

            fire(o, 1)
            drain_pre(1)
            fire_cval(1)

            drain_cval(0)
            acc = compute(0, acc)

            fire(e2, 0)
            drain_pre(0)
            fire_cval(0)

            drain_cval(1)
            acc = compute(1, acc)
            return acc

        acc = lax.fori_loop(0, NCHUNK // 2, pair_body, zf)
        drain_cval(0)

        accv[...] = acc
        pltpu.sync_copy(accv, out.at[wid])

    return sc_weight


def kernel(embeddings, confidence, target, triplets):
    B, D = embeddings.shape
    C = confidence.shape[1]
    T = triplets.shape[0]

    trip_t = triplets.T
    emb_w = lax.bitcast_convert_type(
        embeddings.astype(jnp.bfloat16).reshape(B, D // 2, 2), jnp.int32)

    d1, d2, fidx = _make_sc_dist(B, D, T, 128)(
        emb_w, trip_t, target)

    lse, conf_cm = _make_lse(B, C, 1024)(confidence.T)
    conf_flat = conf_cm.reshape(-1)

    partials = _make_sc_weight(B, T, 512)(
        trip_t, lse, conf_flat, fidx, d1, d2)
    mean = jnp.sum(partials) / jnp.float32(T)
    return (mean, jnp.asarray(T, dtype=jnp.int32))


# speedup vs baseline: 1.0154x; 1.0154x over previous
import functools

import jax
import jax.numpy as jnp
from jax import lax
from jax.experimental import pallas as pl
from jax.experimental.pallas import tpu as pltpu
from jax.experimental.pallas import tpu_sc as plsc

MARGIN1 = 0.4
MARGIN2 = 0.4

NC = 2
NS = 16
LANES = 16
NW = NC * NS


def _make_lse(B, C, RB):
    def body(ct_ref, o_ref, flat_ref):
        x = ct_ref[...]
        m = jnp.max(x, axis=0)
        s = jnp.sum(jnp.exp(x - m[None, :]), axis=0)
        o_ref[...] = m + jnp.log(s)
        for b in range(RB // 128):
            flat_ref[:, b, :] = x[:, b * 128:(b + 1) * 128]

    return pl.pallas_call(
        body,
        grid=(B // RB,),
        in_specs=[pl.BlockSpec((C, RB), lambda i: (0, i))],
        out_specs=[pl.BlockSpec((RB,), lambda i: (i,)),
                   pl.BlockSpec((C, RB // 128, 128), lambda i: (0, i, 0))],
        out_shape=[jax.ShapeDtypeStruct((B,), jnp.float32),
                   jax.ShapeDtypeStruct((C, B // 128, 128), jnp.float32)],
    )


def _make_sc_dist(B, D, T, K):
    TW = T // NW
    NCHUNK = TW // K
    GROUPS = K // LANES

    mesh = plsc.VectorSubcoreMesh(core_axis_name="c", subcore_axis_name="s")

    buf_t = [
        pltpu.VMEM((K,), jnp.int32),
        pltpu.VMEM((K,), jnp.int32),
        pltpu.VMEM((K,), jnp.int32),
        pltpu.VMEM((K,), jnp.int32),
        pltpu.VMEM((K, D), jnp.float32),
        pltpu.VMEM((K, D), jnp.float32),
        pltpu.VMEM((K, D), jnp.float32),
        pltpu.VMEM((K, D), jnp.float32),
        pltpu.VMEM((4 * K,), jnp.int32),
        pltpu.VMEM((K,), jnp.float32),
        pltpu.VMEM((K,), jnp.float32),
    ]

    @functools.partial(
        pl.kernel,
        mesh=mesh,
        compiler_params=pltpu.CompilerParams(
            needs_layout_passes=False, use_tc_tiling_on_sc=False),
        out_type=[jax.ShapeDtypeStruct((T,), jnp.float32),
                  jax.ShapeDtypeStruct((T,), jnp.float32),
                  jax.ShapeDtypeStruct((4 * T,), jnp.int32)],
        scratch_types=[
            pltpu.VMEM((B,), jnp.int32),
            *buf_t,
            *buf_t,
            pltpu.SemaphoreType.DMA,
            pltpu.SemaphoreType.DMA,
            pltpu.SemaphoreType.DMA,
            pltpu.SemaphoreType.DMA,
            pltpu.SemaphoreType.DMA,
        ],
    )
    def sc_dist(emb, tidx, tgt, d1_out, d2_out, fidx_out,
                target_v, *rest):
        bufs = (rest[0:11], rest[11:22])
        sem_i = rest[22]
        sem_d = (rest[23], rest[24])
        sem_o = (rest[25], rest[26])

        wid = lax.axis_index("s") * NC + lax.axis_index("c")
        base_t = wid * TW

        pltpu.sync_copy(tgt, target_v)

        zf = jnp.zeros((LANES,), jnp.float32)
        iota = lax.iota(jnp.int32, LANES)
        def fire_idx(ch, p):
            tb = base_t + ch * K
            for q in range(4):
                pltpu.async_copy(
                    tidx.at[q, pl.ds(tb, K)], bufs[p][q], sem_i)

        def drain_idx(p):
            for q in range(4):
                pltpu.make_async_copy(
                    tidx.at[q, pl.ds(0, K)], bufs[p][q], sem_i).wait()

        def fire_rows(p):
            for q in range(4):
                pltpu.async_copy(emb.at[bufs[p][q]], bufs[p][4 + q], sem_d[p])

        def drain_rows(p):
            for q in range(4):
                pltpu.make_async_copy(
                    emb.at[bufs[p][q]], bufs[p][4 + q], sem_d[p]).wait()

        def phase_a(p, drain_fidx):
            idxa_v, _, idxr_v, idxn_v = bufs[p][0:4]
            fidx_v = bufs[p][8]
            if drain_fidx:
                pltpu.make_async_copy(
                    fidx_v, fidx_out.at[pl.ds(0, 4 * K)], sem_o[p]).wait()

            def ga(g, carry):
                gb = g * LANES
                av = idxa_v[pl.ds(gb, LANES)]
                rv = idxr_v[pl.ds(gb, LANES)]
                nv = idxn_v[pl.ds(gb, LANES)]
                ta = plsc.load_gather(target_v, [av])
                tr = plsc.load_gather(target_v, [rv])
                tn = plsc.load_gather(target_v, [nv])
                fidx_v[pl.ds(gb, LANES)] = tr * B + av
                fidx_v[pl.ds(K + gb, LANES)] = ta * B + rv
                fidx_v[pl.ds(2 * K + gb, LANES)] = tn * B + av
                fidx_v[pl.ds(3 * K + gb, LANES)] = ta * B + nv
                return carry

            lax.fori_loop(0, GROUPS, ga, 0)

        def phase_b(ch, p, drain_d):
            ra_v, rp_v, rr_v, rn_v = bufs[p][4:8]
            d1_v, d2_v = bufs[p][9], bufs[p][10]
            if drain_d:
                pltpu.make_async_copy(
                    d1_v, d1_out.at[pl.ds(0, K)], sem_o[p]).wait()
                pltpu.make_async_copy(
                    d2_v, d2_out.at[pl.ds(0, K)], sem_o[p]).wait()

            def gb_fn(g, carry):
                gb = g * LANES
                jv = gb + iota

                def dbody(dd, c):
                    dap, dar, dan, dv = c
                    for _ in range(8):
                        cv = jnp.bitwise_and(dv + iota, D - 1)
                        ea = plsc.load_gather(ra_v, [jv, cv])
                        ep = plsc.load_gather(rp_v, [jv, cv])
                        er = plsc.load_gather(rr_v, [jv, cv])
                        en = plsc.load_gather(rn_v, [jv, cv])
                        s1 = ea - ep
                        s2 = ea - er
                        s3 = ea - en
                        dap = dap + s1 * s1
                        dar = dar + s2 * s2
                        dan = dan + s3 * s3
                        dv = dv + 1
                    return (dap, dar, dan, dv)

                dap, dar, dan, _ = lax.fori_loop(
                    0, D // 8, dbody,
                    (zf, zf, zf, jnp.zeros((LANES,), jnp.int32)))
                d1_v[pl.ds(gb, LANES)] = dap - dar
                d2_v[pl.ds(gb, LANES)] = dar - dan
                return carry

            lax.fori_loop(0, GROUPS, gb_fn, 0)
            tb = base_t + ch * K
            pltpu.async_copy(d1_v, d1_out.at[pl.ds(tb, K)], sem_o[p])
            pltpu.async_copy(d2_v, d2_out.at[pl.ds(tb, K)], sem_o[p])
            pltpu.async_copy(
                bufs[p][8], fidx_out.at[pl.ds(4 * tb, 4 * K)], sem_o[p])

        fire_idx(0, 0)
        drain_idx(0)
        phase_a(0, False)
        fire_rows(0)

        def pair_impl(e, first):
            o = e + 1
            e2 = jnp.minimum(e + 2, NCHUNK - 2)

            fire_idx(o, 1)
            drain_idx(1)
            phase_a(1, not first)
            fire_rows(1)

            drain_rows(0)
            phase_b(e, 0, not first)

            fire_idx(e2, 0)
            drain_idx(0)
            phase_a(0, True)
            fire_rows(0)

            drain_rows(1)
            phase_b(o, 1, not first)

        pair_impl(0, True)

        def pair_body(gp, carry):
            pair_impl(2 * gp, False)
            return carry

        lax.fori_loop(1, NCHUNK // 2, pair_body, 0)
        drain_rows(0)
        pltpu.make_async_copy(
            bufs[0][9], d1_out.at[pl.ds(0, K)], sem_o[0]).wait()
        pltpu.make_async_copy(
            bufs[0][10], d2_out.at[pl.ds(0, K)], sem_o[0]).wait()
        pltpu.make_async_copy(
            bufs[1][8], fidx_out.at[pl.ds(0, 4 * K)], sem_o[1]).wait()
        pltpu.make_async_copy(
            bufs[1][9], d1_out.at[pl.ds(0, K)], sem_o[1]).wait()
        pltpu.make_async_copy(
            bufs[1][10], d2_out.at[pl.ds(0, K)], sem_o[1]).wait()

    return sc_dist


def _make_sc_weight(B, T, K):
    TW = T // NW
    NCHUNK = TW // K
    GROUPS = K // LANES

    mesh = plsc.VectorSubcoreMesh(core_axis_name="c", subcore_axis_name="s")

    buf_t = [
        pltpu.VMEM((4 * K,), jnp.int32),
        pltpu.VMEM((4 * K,), jnp.float32),
        pltpu.VMEM((K,), jnp.int32),
        pltpu.VMEM((K,), jnp.int32),
        pltpu.VMEM((K,), jnp.int32),
        pltpu.VMEM((K,), jnp.float32),
        pltpu.VMEM((K,), jnp.float32),
    ]

    @functools.partial(
        pl.kernel,
        mesh=mesh,
        compiler_params=pltpu.CompilerParams(
            needs_layout_passes=False, use_tc_tiling_on_sc=False),
        out_type=jax.ShapeDtypeStruct((NW, LANES), jnp.float32),
        scratch_types=[
            pltpu.VMEM((B,), jnp.float32),
            *buf_t,
            *buf_t,
            pltpu.VMEM((LANES,), jnp.float32),
            pltpu.SemaphoreType.DMA,
            pltpu.SemaphoreType.DMA,
        ],
    )
    def sc_weight(tidx, lse, conf_flat, fidx_in, d1_in, d2_in,
                  out, lse_v, *rest):
        bufs = (rest[0:7], rest[7:14])
        accv = rest[14]
        sem = (rest[15], rest[16])

        wid = lax.axis_index("s") * NC + lax.axis_index("c")
        base_t = wid * TW

        pltpu.sync_copy(lse, lse_v)

        zf = jnp.zeros((LANES,), jnp.float32)

        def fire(ch, p):
            tb = base_t + ch * K
            b = bufs[p]
            pltpu.async_copy(fidx_in.at[pl.ds(4 * tb, 4 * K)], b[0], sem[p])
            pltpu.async_copy(tidx.at[0, pl.ds(tb, K)], b[2], sem[p])
            pltpu.async_copy(tidx.at[2, pl.ds(tb, K)], b[3], sem[p])
            pltpu.async_copy(tidx.at[3, pl.ds(tb, K)], b[4], sem[p])
            pltpu.async_copy(d1_in.at[pl.ds(tb, K)], b[5], sem[p])
            pltpu.async_copy(d2_in.at[pl.ds(tb, K)], b[6], sem[p])

        def drain_pre(p):
            b = bufs[p]
            pltpu.make_async_copy(
                fidx_in.at[pl.ds(0, 4 * K)], b[0], sem[p]).wait()
            pltpu.make_async_copy(
                tidx.at[0, pl.ds(0, K)], b[2], sem[p]).wait()
            pltpu.make_async_copy(
                tidx.at[2, pl.ds(0, K)], b[3], sem[p]).wait()
            pltpu.make_async_copy(
                tidx.at[3, pl.ds(0, K)], b[4], sem[p]).wait()
            pltpu.make_async_copy(d1_in.at[pl.ds(0, K)], b[5], sem[p]).wait()
            pltpu.make_async_copy(d2_in.at[pl.ds(0, K)], b[6], sem[p]).wait()

        def fire_cval(p):
            b = bufs[p]
            pltpu.async_copy(conf_flat.at[b[0]], b[1], sem[p])

        def drain_cval(p):
            b = bufs[p]
            pltpu.make_async_copy(conf_flat.at[b[0]], b[1], sem[p]).wait()

        def compute(p, acc):
            b = bufs[p]
            cval_v, idxa_v, idxr_v, idxn_v, d1_v, d2_v = b[1:7]

            def gb_fn(g, acc_in):
                gb = g * LANES
                av = idxa_v[pl.ds(gb, LANES)]
                rv = idxr_v[pl.ds(gb, LANES)]
                nv = idxn_v[pl.ds(gb, LANES)]
                la = plsc.load_gather(lse_v, [av])
                lr = plsc.load_gather(lse_v, [rv])
                ln = plsc.load_gather(lse_v, [nv])
                c1 = cval_v[pl.ds(gb, LANES)]
                c2 = cval_v[pl.ds(K + gb, LANES)]
                c3 = cval_v[pl.ds(2 * K + gb, LANES)]
                c4 = cval_v[pl.ds(3 * K + gb, LANES)]
                w_rel = jnp.exp(jnp.exp(c1 - la) + jnp.exp(c2 - lr))
                w_neg = jnp.exp(jnp.exp(c3 - la) + jnp.exp(c4 - ln))
                d1 = d1_v[pl.ds(gb, LANES)]
                d2 = d2_v[pl.ds(gb, LANES)]
                loss = (jnp.maximum(d1 + w_rel * MARGIN1, 0.0)
                        + jnp.maximum(d2 + w_neg * MARGIN2, 0.0))
                return acc_in + loss

            return lax.fori_loop(0, GROUPS, gb_fn, acc)

        fire(0, 0)
        drain_pre(0)
        fire_cval(0)

        def pair_body(gp, acc):
            e = 2 * gp
            o = e + 1
            e2 = jnp.minimum(e + 2, NCHUNK - 2)

            fire(o, 1)
            drain_pre(1)
            fire_cval(1)

            drain_cval(0)
            acc = compute(0, acc)

            fire(e2, 0)
            drain_pre(0)
            fire_cval(0)

            drain_cval(1)
            acc = compute(1, acc)
            return acc

        acc = lax.fori_loop(0, NCHUNK // 2, pair_body, zf)
        drain_cval(0)

        accv[...] = acc
        pltpu.sync_copy(accv, out.at[wid])

    return sc_weight


def kernel(embeddings, confidence, target, triplets):
    B, D = embeddings.shape
    C = confidence.shape[1]
    T = triplets.shape[0]

    trip_t = triplets.T

    d1, d2, fidx = _make_sc_dist(B, D, T, 128)(
        embeddings, trip_t, target)

    lse, conf_cm = _make_lse(B, C, 1024)(confidence.T)
    conf_flat = conf_cm.reshape(-1)

    partials = _make_sc_weight(B, T, 512)(
        trip_t, lse, conf_flat, fidx, d1, d2)
    mean = jnp.sum(partials) / jnp.float32(T)
    return (mean, jnp.asarray(T, dtype=jnp.int32))
